# baseline (speedup 1.0000x reference)
import functools
import math

import jax
import jax.numpy as jnp
from jax import lax
from jax.experimental import pallas as pl
from jax.experimental.pallas import tpu as pltpu
from jax.experimental.pallas import tpu_sc as plsc

N = 10000
NPASS = 2
NH = 5120
NPAD = NPASS * NH
E = 160000
DH = 32
HALF = 128
CHUNK = 80
IDXB = 8
NS = 16
NPS = NH // NS
ZPS = 16
NZROW = NS * ZPS
EPS_SUB = E // NS
NCHUNK = EPS_SUB // CHUNK
NBLK = 4
BROW = NPS // NBLK
INV_SCALE = 1.0 / math.sqrt(DH)
EPS = 1e-6


def _sc_attention(qh, kh, vh, src, dst, zrows):
    mesh = plsc.VectorSubcoreMesh(core_axis_name="c", subcore_axis_name="s")

    @functools.partial(
        pl.kernel,
        out_type=jax.ShapeDtypeStruct((NPAD, 2 * HALF), jnp.float32),
        mesh=mesh,
        compiler_params=pltpu.CompilerParams(needs_layout_passes=False),
        scratch_types=[
            pltpu.VMEM((IDXB * CHUNK,), jnp.int32),
            pltpu.VMEM((IDXB * CHUNK,), jnp.int32),
            pltpu.VMEM((CHUNK,), jnp.int32),
            pltpu.VMEM((CHUNK,), jnp.int32),
            pltpu.VMEM((CHUNK, HALF), jnp.float32),
            pltpu.VMEM((CHUNK, HALF), jnp.float32),
            pltpu.VMEM((CHUNK, HALF), jnp.float32),
            pltpu.VMEM((CHUNK, HALF), jnp.float32),
            pltpu.VMEM((ZPS, HALF), jnp.float32),
            pltpu.VMEM_SHARED((NH, HALF), jnp.float32),
            pltpu.VMEM_SHARED((NZROW, HALF), jnp.float32),
            pltpu.SemaphoreType.DMA,
            pltpu.SemaphoreType.DMA,
            pltpu.SemaphoreType.DMA,
            pltpu.SemaphoreType.DMA,
            pltpu.SemaphoreType.DMA,
        ],
    )
    def attn(qh_h, kh_h, vh_h, src_h, dst_h, z_h, out_h,
             sidxb, didxb, lidx, zridx, kbuf, qbuf, vbuf, zbuf,
             zstage, acc, accz,
             sem0, sem1, sem2, sem3, sem4):
        stage = kbuf
        outb = qbuf
        c = lax.axis_index("c")
        s = lax.axis_index("s")
        coff = pl.multiple_of(c * HALF, HALF)
        kh_c = kh_h.at[c]
        qh_c = qh_h.at[c]
        vh_c = vh_h.at[c]
        ebase = s * EPS_SUB
        lane = lax.iota(jnp.int32, 16)

        def zero_accs():
            pltpu.sync_copy(z_h, acc.at[pl.ds(s * NPS, NPS)])

            @pl.when(s < NZROW // 64)
            def _():
                pltpu.sync_copy(z_h.at[pl.ds(0, 64)],
                                accz.at[pl.ds(s * 64, 64)])

        zero_accs()

        def zb_body(e, carry):
            for j in range(HALF // 16):
                zbuf[e, pl.ds(j * 16, 16)] = jnp.zeros((16,), jnp.float32)
            return carry

        lax.fori_loop(0, CHUNK, zb_body, 0)
        plsc.subcore_barrier()

        BLK = IDXB * CHUNK

        def load_idx_block(b):
            eb = ebase + b * BLK
            pltpu.sync_copy(src_h.at[pl.ds(eb, BLK)], sidxb)
            pltpu.sync_copy(dst_h.at[pl.ds(eb, BLK)], didxb)

        def gather_kq(i):
            off = (i % IDXB) * CHUNK
            pltpu.async_copy(kh_c.at[sidxb.at[pl.ds(off, CHUNK)]], kbuf, sem0)
            pltpu.async_copy(qh_c.at[didxb.at[pl.ds(off, CHUNK)]], qbuf, sem1)

        def gather_v(i):
            off = (i % IDXB) * CHUNK
            pltpu.async_copy(vh_c.at[sidxb.at[pl.ds(off, CHUNK)]], vbuf, sem2)

        def wait_gathers():
            pltpu.make_async_copy(kh_c.at[sidxb.at[pl.ds(0, CHUNK)]],
                                  kbuf, sem0).wait()
            pltpu.make_async_copy(qh_c.at[didxb.at[pl.ds(0, CHUNK)]],
                                  qbuf, sem1).wait()
            pltpu.make_async_copy(vh_c.at[sidxb.at[pl.ds(0, CHUNK)]],
                                  vbuf, sem2).wait()

        def run_pass(p):
            nlo = p * NH
            load_idx_block(0)
            gather_kq(0)
            gather_v(0)

            def chunk_body(i, carry):
                off = (i % IDXB) * CHUNK
                wait_gathers()

                def group_body(g, carry2):
                    eidx = g * 16 + lane
                    dv = didxb[pl.ds(off + g * 16, 16)]
                    inr = jnp.logical_and(dv >= nlo, dv < nlo + NH)
                    lv = jnp.clip(dv - nlo, 0, NH - 1)
                    zr = (lv // NPS) * ZPS + (lv % NPS) // 32
                    zc = ((lv % NPS) % 32) * 4
                    plsc.store_scatter(lidx, [eidx], lv)
                    plsc.store_scatter(zridx, [eidx], zr)
                    for h in range(4):
                        o = h * DH
                        acc_h = jnp.zeros((16,), jnp.float32)
                        for d in range(DH):
                            col = jnp.full((16,), o + d, jnp.int32)
                            kv = plsc.load_gather(kbuf, [eidx, col])
                            qv = plsc.load_gather(qbuf, [eidx, col])
                            acc_h = acc_h + kv * qv
                        sh = jnp.clip(acc_h * INV_SCALE, -5.0, 5.0)
                        pv = jnp.exp(sh)
                        pm = jnp.where(inr, pv, 0.0)
                        for d in range(DH):
                            col = jnp.full((16,), o + d, jnp.int32)
                            mv = plsc.load_gather(vbuf, [eidx, col]) * pm
                            plsc.store_scatter(vbuf, [eidx, col], mv)
                        plsc.store_scatter(zbuf, [eidx, zc + h], pm)
                    return carry2

                lax.fori_loop(0, CHUNK // 16, group_body, 0)
                pltpu.async_copy(vbuf, acc.at[lidx], sem3, add=True)
                pltpu.async_copy(zbuf, accz.at[zridx], sem4, add=True)

                pltpu.make_async_copy(zbuf, accz.at[zridx], sem4).wait()

                def unz_body(g, carry2):
                    eidx = g * 16 + lane
                    dv = didxb[pl.ds(off + g * 16, 16)]
                    lv = jnp.clip(dv - nlo, 0, NH - 1)
                    zc = ((lv % NPS) % 32) * 4
                    zero = jnp.zeros((16,), jnp.float32)
                    for h in range(4):
                        plsc.store_scatter(zbuf, [eidx, zc + h], zero)
                    return carry2

                lax.fori_loop(0, CHUNK // 16, unz_body, 0)

                @pl.when((i + 1) % IDXB == 0)
                def _():
                    load_idx_block((i + 1) // IDXB)

                gather_kq(i + 1)
                pltpu.make_async_copy(vbuf, acc.at[lidx], sem3).wait()
                gather_v(i + 1)
                return carry

            lax.fori_loop(0, NCHUNK, chunk_body, 0)
            wait_gathers()
            plsc.subcore_barrier()

            nb = s * NPS
            pltpu.sync_copy(accz.at[pl.ds(s * ZPS, ZPS)], zstage)

            def norm_body(j, carry):
                rb = nb + j * BROW
                pltpu.sync_copy(acc.at[pl.ds(rb, BROW)], stage)

                def grp_body(g, carry2):
                    lnode = j * BROW + g * 16 + lane
                    nl = g * 16 + lane
                    zr = lnode // 32
                    zc0 = (lnode % 32) * 4
                    for h in range(4):
                        o = h * DH
                        zv = plsc.load_gather(zstage, [zr, zc0 + h])
                        rcp = 1.0 / (zv + EPS)
                        for d in range(DH):
                            col = jnp.full((16,), o + d, jnp.int32)
                            wv = plsc.load_gather(stage, [nl, col])
                            plsc.store_scatter(outb, [nl, col], wv * rcp)
                    return carry2

                lax.fori_loop(0, BROW // 16, grp_body, 0)
                pltpu.sync_copy(outb, out_h.at[pl.ds(nlo + rb, BROW), pl.ds(coff, HALF)])
                return carry

            lax.fori_loop(0, NBLK, norm_body, 0)

        run_pass(0)
        for p in range(1, NPASS):
            plsc.subcore_barrier()
            zero_accs()
            plsc.subcore_barrier()
            run_pass(p)

    return attn(qh, kh, vh, src, dst, zrows)


def kernel(q, k, v, edge_index):
    q2 = q.reshape(N, 2, HALF).transpose(1, 0, 2)
    k2 = k.reshape(N, 2, HALF).transpose(1, 0, 2)
    v2 = v.reshape(N, 2, HALF).transpose(1, 0, 2)
    pad = jnp.zeros((IDXB * CHUNK,), jnp.int32)
    src = jnp.concatenate([edge_index[0].astype(jnp.int32), pad])
    dst = jnp.concatenate([edge_index[1].astype(jnp.int32), pad])
    zrows = jnp.zeros((NPS, HALF), jnp.float32)
    out2 = _sc_attention(q2, k2, v2, src, dst, zrows)
    return out2[:N].reshape(1, N, 2 * HALF)


# speedup vs baseline: 1.0350x; 1.0350x over previous
import functools
import math

import jax
import jax.numpy as jnp
from jax import lax
from jax.experimental import pallas as pl
from jax.experimental.pallas import tpu as pltpu
from jax.experimental.pallas import tpu_sc as plsc

N = 10000
NPASS = 2
NH = 5120
NPAD = NPASS * NH
E = 160000
DH = 32
HALF = 128
CHUNK = 80
IDXB = 8
NS = 16
NPS = NH // NS
ZPS = 16
NZROW = NS * ZPS
EPS_SUB = E // NS
NCHUNK = EPS_SUB // CHUNK
NBLK = 4
BROW = NPS // NBLK
INV_SCALE = 1.0 / math.sqrt(DH)
EPS = 1e-6


def _sc_attention(qh, kh, vh, src, dst, zrows, zzrows):
    mesh = plsc.VectorSubcoreMesh(core_axis_name="c", subcore_axis_name="s")

    @functools.partial(
        pl.kernel,
        out_type=jax.ShapeDtypeStruct((NPAD, 2 * HALF), jnp.float32),
        mesh=mesh,
        compiler_params=pltpu.CompilerParams(needs_layout_passes=False),
        scratch_types=[
            pltpu.VMEM((IDXB * CHUNK,), jnp.int32),
            pltpu.VMEM((IDXB * CHUNK,), jnp.int32),
            pltpu.VMEM((CHUNK,), jnp.int32),
            pltpu.VMEM((CHUNK,), jnp.int32),
            pltpu.VMEM((CHUNK, HALF), jnp.float32),
            pltpu.VMEM((CHUNK, HALF), jnp.float32),
            pltpu.VMEM((CHUNK, HALF), jnp.float32),
            pltpu.VMEM((4 * CHUNK,), jnp.float32),
            pltpu.VMEM((4 * CHUNK,), jnp.int32),
            pltpu.VMEM((4 * NPS,), jnp.float32),
            pltpu.VMEM_SHARED((NH, HALF), jnp.float32),
            pltpu.VMEM_SHARED((4 * NH,), jnp.float32),
            pltpu.SemaphoreType.DMA,
            pltpu.SemaphoreType.DMA,
            pltpu.SemaphoreType.DMA,
            pltpu.SemaphoreType.DMA,
            pltpu.SemaphoreType.DMA,
        ],
    )
    def attn(qh_h, kh_h, vh_h, src_h, dst_h, z_h, zz_h, out_h,
             sidxb, didxb, lidx, zridx, kbuf, qbuf, vbuf,
             zvals, zidx, zstage, acc, accz,
             sem0, sem1, sem2, sem3, sem4):
        stage = kbuf
        outb = qbuf
        c = lax.axis_index("c")
        s = lax.axis_index("s")
        coff = pl.multiple_of(c * HALF, HALF)
        kh_c = kh_h.at[c]
        qh_c = qh_h.at[c]
        vh_c = vh_h.at[c]
        ebase = s * EPS_SUB
        lane = lax.iota(jnp.int32, 16)

        def zero_accs():
            pltpu.sync_copy(z_h, acc.at[pl.ds(s * NPS, NPS)])
            pltpu.sync_copy(zz_h, accz.at[pl.ds(s * 4 * NPS, 4 * NPS)])

        zero_accs()
        plsc.subcore_barrier()

        BLK = IDXB * CHUNK

        def load_idx_block(b):
            eb = ebase + b * BLK
            pltpu.sync_copy(src_h.at[pl.ds(eb, BLK)], sidxb)
            pltpu.sync_copy(dst_h.at[pl.ds(eb, BLK)], didxb)

        def gather_kq(i):
            off = (i % IDXB) * CHUNK
            pltpu.async_copy(kh_c.at[sidxb.at[pl.ds(off, CHUNK)]], kbuf, sem0)
            pltpu.async_copy(qh_c.at[didxb.at[pl.ds(off, CHUNK)]], qbuf, sem1)

        def gather_v(i):
            off = (i % IDXB) * CHUNK
            pltpu.async_copy(vh_c.at[sidxb.at[pl.ds(off, CHUNK)]], vbuf, sem2)

        def wait_gathers():
            pltpu.make_async_copy(kh_c.at[sidxb.at[pl.ds(0, CHUNK)]],
                                  kbuf, sem0).wait()
            pltpu.make_async_copy(qh_c.at[didxb.at[pl.ds(0, CHUNK)]],
                                  qbuf, sem1).wait()
            pltpu.make_async_copy(vh_c.at[sidxb.at[pl.ds(0, CHUNK)]],
                                  vbuf, sem2).wait()

        def run_pass(p):
            nlo = p * NH
            load_idx_block(0)
            gather_kq(0)
            gather_v(0)

            def chunk_body(i, carry):
                off = (i % IDXB) * CHUNK
                wait_gathers()

                def group_body(g, carry2):
                    eidx = g * 16 + lane
                    dv = didxb[pl.ds(off + g * 16, 16)]
                    inr = jnp.logical_and(dv >= nlo, dv < nlo + NH)
                    lv = jnp.clip(dv - nlo, 0, NH - 1)
                    plsc.store_scatter(lidx, [eidx], lv)
                    for h in range(4):
                        o = h * DH
                        acc_h = jnp.zeros((16,), jnp.float32)
                        for d in range(DH):
                            col = jnp.full((16,), o + d, jnp.int32)
                            kv = plsc.load_gather(kbuf, [eidx, col])
                            qv = plsc.load_gather(qbuf, [eidx, col])
                            acc_h = acc_h + kv * qv
                        sh = jnp.clip(acc_h * INV_SCALE, -5.0, 5.0)
                        pv = jnp.exp(sh)
                        pm = jnp.where(inr, pv, 0.0)
                        for d in range(DH):
                            col = jnp.full((16,), o + d, jnp.int32)
                            mv = plsc.load_gather(vbuf, [eidx, col]) * pm
                            plsc.store_scatter(vbuf, [eidx, col], mv)
                        e4 = eidx * 4 + h
                        plsc.store_scatter(zvals, [e4], pm)
                        plsc.store_scatter(zidx, [e4], lv * 4 + h)
                    return carry2

                lax.fori_loop(0, CHUNK // 16, group_body, 0)
                pltpu.async_copy(vbuf, acc.at[lidx], sem3, add=True)
                pltpu.async_copy(zvals, accz.at[zidx], sem4, add=True)

                @pl.when((i + 1) % IDXB == 0)
                def _():
                    load_idx_block((i + 1) // IDXB)

                gather_kq(i + 1)
                pltpu.make_async_copy(vbuf, acc.at[lidx], sem3).wait()
                pltpu.make_async_copy(zvals, accz.at[zidx], sem4).wait()
                gather_v(i + 1)
                return carry

            lax.fori_loop(0, NCHUNK, chunk_body, 0)
            wait_gathers()
            plsc.subcore_barrier()

            nb = s * NPS
            pltpu.sync_copy(accz.at[pl.ds(s * 4 * NPS, 4 * NPS)], zstage)

            def norm_body(j, carry):
                rb = nb + j * BROW
                pltpu.sync_copy(acc.at[pl.ds(rb, BROW)], stage)

                def grp_body(g, carry2):
                    lnode = j * BROW + g * 16 + lane
                    nl = g * 16 + lane
                    for h in range(4):
                        o = h * DH
                        zv = plsc.load_gather(zstage, [lnode * 4 + h])
                        rcp = 1.0 / (zv + EPS)
                        for d in range(DH):
                            col = jnp.full((16,), o + d, jnp.int32)
                            wv = plsc.load_gather(stage, [nl, col])
                            plsc.store_scatter(outb, [nl, col], wv * rcp)
                    return carry2

                lax.fori_loop(0, BROW // 16, grp_body, 0)
                pltpu.sync_copy(outb, out_h.at[pl.ds(nlo + rb, BROW), pl.ds(coff, HALF)])
                return carry

            lax.fori_loop(0, NBLK, norm_body, 0)

        run_pass(0)
        for p in range(1, NPASS):
            plsc.subcore_barrier()
            zero_accs()
            plsc.subcore_barrier()
            run_pass(p)

    return attn(qh, kh, vh, src, dst, zrows, zzrows)


def kernel(q, k, v, edge_index):
    q2 = q.reshape(N, 2, HALF).transpose(1, 0, 2)
    k2 = k.reshape(N, 2, HALF).transpose(1, 0, 2)
    v2 = v.reshape(N, 2, HALF).transpose(1, 0, 2)
    pad = jnp.zeros((IDXB * CHUNK,), jnp.int32)
    src = jnp.concatenate([edge_index[0].astype(jnp.int32), pad])
    dst = jnp.concatenate([edge_index[1].astype(jnp.int32), pad])
    zrows = jnp.zeros((NPS, HALF), jnp.float32)
    zzrows = jnp.zeros((4 * NPS,), jnp.float32)
    out2 = _sc_attention(q2, k2, v2, src, dst, zrows, zzrows)
    return out2[:N].reshape(1, N, 2 * HALF)


# speedup vs baseline: 1.0378x; 1.0027x over previous
import functools
import math

import jax
import jax.numpy as jnp
from jax import lax
from jax.experimental import pallas as pl
from jax.experimental.pallas import tpu as pltpu
from jax.experimental.pallas import tpu_sc as plsc

N = 10000
NPASS = 2
NH = 5120
NPAD = NPASS * NH
E = 160000
DH = 32
HALF = 128
CHUNK = 80
IDXB = 8
NS = 16
NPS = NH // NS
ZPS = 16
NZROW = NS * ZPS
EPS_SUB = E // NS
NCHUNK = EPS_SUB // CHUNK
NBLK = 4
BROW = NPS // NBLK
INV_SCALE = 1.0 / math.sqrt(DH)
EPS = 1e-6


def _sc_attention(qh, kh, vh, src, dst, zrows, zzrows):
    mesh = plsc.VectorSubcoreMesh(core_axis_name="c", subcore_axis_name="s")

    @functools.partial(
        pl.kernel,
        out_type=jax.ShapeDtypeStruct((NPAD, 2 * HALF), jnp.float32),
        mesh=mesh,
        compiler_params=pltpu.CompilerParams(needs_layout_passes=False),
        scratch_types=[
            pltpu.VMEM((IDXB * CHUNK,), jnp.int32),
            pltpu.VMEM((IDXB * CHUNK,), jnp.int32),
            pltpu.VMEM((CHUNK,), jnp.int32),
            pltpu.VMEM((CHUNK,), jnp.int32),
            pltpu.VMEM((CHUNK, HALF), jnp.float32),
            pltpu.VMEM((CHUNK, HALF), jnp.float32),
            pltpu.VMEM((CHUNK, HALF), jnp.float32),
            pltpu.VMEM((CHUNK, HALF), jnp.float32),
            pltpu.VMEM((4 * CHUNK,), jnp.float32),
            pltpu.VMEM((4 * CHUNK,), jnp.int32),
            pltpu.VMEM((4 * NPS,), jnp.float32),
            pltpu.VMEM_SHARED((NH, HALF), jnp.float32),
            pltpu.VMEM_SHARED((4 * NH,), jnp.float32),
            pltpu.SemaphoreType.DMA,
            pltpu.SemaphoreType.DMA,
            pltpu.SemaphoreType.DMA,
            pltpu.SemaphoreType.DMA,
            pltpu.SemaphoreType.DMA,
        ],
    )
    def attn(qh_h, kh_h, vh_h, src_h, dst_h, z_h, zz_h, out_h,
             sidxb, didxb, lidx, zridx, kbuf, qbuf, vbuf, msgbuf,
             zvals, zidx, zstage, acc, accz,
             sem0, sem1, sem2, sem3, sem4):
        stage = kbuf
        outb = qbuf
        c = lax.axis_index("c")
        s = lax.axis_index("s")
        coff = pl.multiple_of(c * HALF, HALF)
        kh_c = kh_h.at[c]
        qh_c = qh_h.at[c]
        vh_c = vh_h.at[c]
        ebase = s * EPS_SUB
        lane = lax.iota(jnp.int32, 16)

        def zero_accs():
            pltpu.sync_copy(z_h, acc.at[pl.ds(s * NPS, NPS)])
            pltpu.sync_copy(zz_h, accz.at[pl.ds(s * 4 * NPS, 4 * NPS)])

        zero_accs()
        plsc.subcore_barrier()

        BLK = IDXB * CHUNK

        def load_idx_block(b):
            eb = ebase + b * BLK
            pltpu.sync_copy(src_h.at[pl.ds(eb, BLK)], sidxb)
            pltpu.sync_copy(dst_h.at[pl.ds(eb, BLK)], didxb)

        def gather_kq(i):
            off = (i % IDXB) * CHUNK
            pltpu.async_copy(kh_c.at[sidxb.at[pl.ds(off, CHUNK)]], kbuf, sem0)
            pltpu.async_copy(qh_c.at[didxb.at[pl.ds(off, CHUNK)]], qbuf, sem1)

        def gather_v(i):
            off = (i % IDXB) * CHUNK
            pltpu.async_copy(vh_c.at[sidxb.at[pl.ds(off, CHUNK)]], vbuf, sem2)

        def wait_gathers():
            pltpu.make_async_copy(kh_c.at[sidxb.at[pl.ds(0, CHUNK)]],
                                  kbuf, sem0).wait()
            pltpu.make_async_copy(qh_c.at[didxb.at[pl.ds(0, CHUNK)]],
                                  qbuf, sem1).wait()
            pltpu.make_async_copy(vh_c.at[sidxb.at[pl.ds(0, CHUNK)]],
                                  vbuf, sem2).wait()

        def run_pass(p):
            nlo = p * NH
            load_idx_block(0)
            gather_kq(0)
            gather_v(0)

            def chunk_body(i, carry):
                off = (i % IDXB) * CHUNK
                wait_gathers()

                @pl.when(i > 0)
                def _():
                    pltpu.make_async_copy(msgbuf, acc.at[lidx], sem3).wait()
                    pltpu.make_async_copy(zvals, accz.at[zidx], sem4).wait()

                def group_body(g, carry2):
                    eidx = g * 16 + lane
                    dv = didxb[pl.ds(off + g * 16, 16)]
                    inr = jnp.logical_and(dv >= nlo, dv < nlo + NH)
                    lv = jnp.clip(dv - nlo, 0, NH - 1)
                    plsc.store_scatter(lidx, [eidx], lv)
                    for h in range(4):
                        o = h * DH
                        acc_h = jnp.zeros((16,), jnp.float32)
                        for d in range(DH):
                            col = jnp.full((16,), o + d, jnp.int32)
                            kv = plsc.load_gather(kbuf, [eidx, col])
                            qv = plsc.load_gather(qbuf, [eidx, col])
                            acc_h = acc_h + kv * qv
                        sh = jnp.clip(acc_h * INV_SCALE, -5.0, 5.0)
                        pv = jnp.exp(sh)
                        pm = jnp.where(inr, pv, 0.0)
                        for d in range(DH):
                            col = jnp.full((16,), o + d, jnp.int32)
                            mv = plsc.load_gather(vbuf, [eidx, col]) * pm
                            plsc.store_scatter(msgbuf, [eidx, col], mv)
                        e4 = eidx * 4 + h
                        plsc.store_scatter(zvals, [e4], pm)
                        plsc.store_scatter(zidx, [e4], lv * 4 + h)
                    return carry2

                lax.fori_loop(0, CHUNK // 16, group_body, 0)
                pltpu.async_copy(msgbuf, acc.at[lidx], sem3, add=True)
                pltpu.async_copy(zvals, accz.at[zidx], sem4, add=True)

                @pl.when((i + 1) % IDXB == 0)
                def _():
                    load_idx_block((i + 1) // IDXB)

                gather_kq(i + 1)
                gather_v(i + 1)
                return carry

            lax.fori_loop(0, NCHUNK, chunk_body, 0)
            wait_gathers()
            pltpu.make_async_copy(msgbuf, acc.at[lidx], sem3).wait()
            pltpu.make_async_copy(zvals, accz.at[zidx], sem4).wait()
            plsc.subcore_barrier()

            nb = s * NPS
            pltpu.sync_copy(accz.at[pl.ds(s * 4 * NPS, 4 * NPS)], zstage)

            def norm_body(j, carry):
                rb = nb + j * BROW
                pltpu.sync_copy(acc.at[pl.ds(rb, BROW)], stage)

                def grp_body(g, carry2):
                    lnode = j * BROW + g * 16 + lane
                    nl = g * 16 + lane
                    for h in range(4):
                        o = h * DH
                        zv = plsc.load_gather(zstage, [lnode * 4 + h])
                        rcp = 1.0 / (zv + EPS)
                        for d in range(DH):
                            col = jnp.full((16,), o + d, jnp.int32)
                            wv = plsc.load_gather(stage, [nl, col])
                            plsc.store_scatter(outb, [nl, col], wv * rcp)
                    return carry2

                lax.fori_loop(0, BROW // 16, grp_body, 0)
                pltpu.sync_copy(outb, out_h.at[pl.ds(nlo + rb, BROW), pl.ds(coff, HALF)])
                return carry

            lax.fori_loop(0, NBLK, norm_body, 0)

        run_pass(0)
        for p in range(1, NPASS):
            plsc.subcore_barrier()
            zero_accs()
            plsc.subcore_barrier()
            run_pass(p)

    return attn(qh, kh, vh, src, dst, zrows, zzrows)


def kernel(q, k, v, edge_index):
    q2 = q.reshape(N, 2, HALF).transpose(1, 0, 2)
    k2 = k.reshape(N, 2, HALF).transpose(1, 0, 2)
    v2 = v.reshape(N, 2, HALF).transpose(1, 0, 2)
    pad = jnp.zeros((IDXB * CHUNK,), jnp.int32)
    src = jnp.concatenate([edge_index[0].astype(jnp.int32), pad])
    dst = jnp.concatenate([edge_index[1].astype(jnp.int32), pad])
    zrows = jnp.zeros((NPS, HALF), jnp.float32)
    zzrows = jnp.zeros((4 * NPS,), jnp.float32)
    out2 = _sc_attention(q2, k2, v2, src, dst, zrows, zzrows)
    return out2[:N].reshape(1, N, 2 * HALF)


# speedup vs baseline: 1.3179x; 1.2699x over previous
import functools
import math

import jax
import jax.numpy as jnp
from jax import lax
from jax.experimental import pallas as pl
from jax.experimental.pallas import tpu as pltpu
from jax.experimental.pallas import tpu_sc as plsc

N = 10000
NPASS = 2
NH = 5120
NPAD = NPASS * NH
E = 160000
DH = 32
HALF = 128
CHUNK = 80
IDXB = 8
NS = 16
NPS = NH // NS
EPS_SUB = E // NS
NCHUNK = EPS_SUB // CHUNK
NBLK = 4
BROW = NPS // NBLK
PCN = 4 * CHUNK
PCS = 384
INV_SCALE = 1.0 / math.sqrt(DH)
EPS = 1e-6


def _sc_attention(qh, kh, vh, src, dst, zrows, zzrows):
    mesh = plsc.VectorSubcoreMesh(core_axis_name="c", subcore_axis_name="s")

    @functools.partial(
        pl.kernel,
        out_type=(
            jax.ShapeDtypeStruct((NPAD, 2 * HALF), jnp.float32),
            jax.ShapeDtypeStruct((2, (NS * NCHUNK + 1) * PCS), jnp.float32),
        ),
        mesh=mesh,
        compiler_params=pltpu.CompilerParams(needs_layout_passes=False),
        scratch_types=[
            pltpu.VMEM((IDXB * CHUNK,), jnp.int32),
            pltpu.VMEM((IDXB * CHUNK,), jnp.int32),
            pltpu.VMEM((CHUNK,), jnp.int32),
            pltpu.VMEM((CHUNK, HALF), jnp.float32),
            pltpu.VMEM((CHUNK, HALF), jnp.float32),
            pltpu.VMEM((CHUNK, HALF), jnp.float32),
            pltpu.VMEM((CHUNK, HALF), jnp.float32),
            pltpu.VMEM((4 * CHUNK,), jnp.float32),
            pltpu.VMEM((4 * CHUNK,), jnp.int32),
            pltpu.VMEM((2 * PCS,), jnp.float32),
            pltpu.VMEM((4 * NPS,), jnp.float32),
            pltpu.VMEM_SHARED((NH, HALF), jnp.float32),
            pltpu.VMEM_SHARED((4 * NH,), jnp.float32),
            pltpu.SemaphoreType.DMA,
            pltpu.SemaphoreType.DMA,
            pltpu.SemaphoreType.DMA,
            pltpu.SemaphoreType.DMA,
            pltpu.SemaphoreType.DMA,
            pltpu.SemaphoreType.DMA,
        ],
    )
    def attn(qh_h, kh_h, vh_h, src_h, dst_h, z_h, zz_h, out_h, pc_h,
             sidxb, didxb, lidx, kbuf, qbuf, vbuf, msgbuf,
             zvals, zidx, pvbuf, zstage, acc, accz,
             sem0, sem1, sem2, sem3, sem4, sem5):
        stage = kbuf
        outb = qbuf
        c = lax.axis_index("c")
        s = lax.axis_index("s")
        coff = pl.multiple_of(c * HALF, HALF)
        kh_c = kh_h.at[c]
        qh_c = qh_h.at[c]
        vh_c = vh_h.at[c]
        pc_c = pc_h.at[c]
        ebase = s * EPS_SUB
        lane = lax.iota(jnp.int32, 16)
        BLK = IDXB * CHUNK

        def zero_accs():
            pltpu.sync_copy(z_h, acc.at[pl.ds(s * NPS, NPS)])
            pltpu.sync_copy(zz_h, accz.at[pl.ds(s * 4 * NPS, 4 * NPS)])

        zero_accs()
        plsc.subcore_barrier()

        def load_idx_block(b):
            eb = ebase + b * BLK
            pltpu.sync_copy(src_h.at[pl.ds(eb, BLK)], sidxb)
            pltpu.sync_copy(dst_h.at[pl.ds(eb, BLK)], didxb)

        def gather_kq(i):
            off = (i % IDXB) * CHUNK
            pltpu.async_copy(kh_c.at[sidxb.at[pl.ds(off, CHUNK)]], kbuf, sem0)
            pltpu.async_copy(qh_c.at[didxb.at[pl.ds(off, CHUNK)]], qbuf, sem1)

        def gather_v(i):
            off = (i % IDXB) * CHUNK
            pltpu.async_copy(vh_c.at[sidxb.at[pl.ds(off, CHUNK)]], vbuf, sem2)

        def fetch_pv(i):
            slot = (s * NCHUNK + i) * PCS
            pltpu.async_copy(pc_c.at[pl.ds(slot, PCS)],
                             pvbuf.at[pl.ds((i % 2) * PCS, PCS)], sem5)

        def wait_pv(i):
            slot = (s * NCHUNK + i) * PCS
            pltpu.make_async_copy(pc_c.at[pl.ds(slot, PCS)],
                                  pvbuf.at[pl.ds((i % 2) * PCS, PCS)],
                                  sem5).wait()

        def run_pass(p):
            nlo = p * NH
            load_idx_block(0)
            if p == 0:
                gather_kq(0)
            else:
                fetch_pv(0)
            gather_v(0)

            def chunk_body(i, carry):
                off = (i % IDXB) * CHUNK
                slot = (s * NCHUNK + i) * PCS
                if p == 0:
                    pltpu.make_async_copy(
                        kh_c.at[sidxb.at[pl.ds(0, CHUNK)]], kbuf, sem0).wait()
                    pltpu.make_async_copy(
                        qh_c.at[didxb.at[pl.ds(0, CHUNK)]], qbuf, sem1).wait()
                else:
                    wait_pv(i)
                pltpu.make_async_copy(
                    vh_c.at[sidxb.at[pl.ds(0, CHUNK)]], vbuf, sem2).wait()

                @pl.when(i > 0)
                def _():
                    pltpu.make_async_copy(msgbuf, acc.at[lidx], sem3).wait()
                    pltpu.make_async_copy(zvals, accz.at[zidx], sem4).wait()
                    if p == 0:
                        pltpu.make_async_copy(
                            pvbuf.at[pl.ds(0, PCS)],
                            pc_c.at[pl.ds(slot, PCS)], sem5).wait()

                def group_body(g, carry2):
                    eidx = g * 16 + lane
                    dv = didxb[pl.ds(off + g * 16, 16)]
                    inr = jnp.logical_and(dv >= nlo, dv < nlo + NH)
                    lv = jnp.clip(dv - nlo, 0, NH - 1)
                    plsc.store_scatter(lidx, [eidx], lv)
                    for h in range(4):
                        o = h * DH
                        e4 = eidx * 4 + h
                        if p == 0:
                            acc_h = jnp.zeros((16,), jnp.float32)
                            for d in range(DH):
                                col = jnp.full((16,), o + d, jnp.int32)
                                kv = plsc.load_gather(kbuf, [eidx, col])
                                qv = plsc.load_gather(qbuf, [eidx, col])
                                acc_h = acc_h + kv * qv
                            sh = jnp.clip(acc_h * INV_SCALE, -5.0, 5.0)
                            pv = jnp.exp(sh)
                            plsc.store_scatter(pvbuf, [e4], pv)
                        else:
                            pv = plsc.load_gather(
                                pvbuf, [(i % 2) * PCS + e4])
                        pm = jnp.where(inr, pv, 0.0)
                        for d in range(DH):
                            col = jnp.full((16,), o + d, jnp.int32)
                            mv = plsc.load_gather(vbuf, [eidx, col]) * pm
                            plsc.store_scatter(msgbuf, [eidx, col], mv)
                        plsc.store_scatter(zvals, [e4], pm)
                        plsc.store_scatter(zidx, [e4], lv * 4 + h)
                    return carry2

                lax.fori_loop(0, CHUNK // 16, group_body, 0)
                pltpu.async_copy(msgbuf, acc.at[lidx], sem3, add=True)
                pltpu.async_copy(zvals, accz.at[zidx], sem4, add=True)
                if p == 0:
                    pltpu.async_copy(pvbuf.at[pl.ds(0, PCS)],
                                     pc_c.at[pl.ds(slot, PCS)], sem5)

                @pl.when((i + 1) % IDXB == 0)
                def _():
                    load_idx_block((i + 1) // IDXB)

                if p == 0:
                    gather_kq(i + 1)
                else:
                    fetch_pv(i + 1)
                gather_v(i + 1)
                return carry

            lax.fori_loop(0, NCHUNK, chunk_body, 0)
            if p == 0:
                pltpu.make_async_copy(
                    kh_c.at[sidxb.at[pl.ds(0, CHUNK)]], kbuf, sem0).wait()
                pltpu.make_async_copy(
                    qh_c.at[didxb.at[pl.ds(0, CHUNK)]], qbuf, sem1).wait()
                pltpu.make_async_copy(
                    pvbuf.at[pl.ds(0, PCS)],
                    pc_c.at[pl.ds(0, PCS)], sem5).wait()
            else:
                wait_pv(NCHUNK)
            pltpu.make_async_copy(
                vh_c.at[sidxb.at[pl.ds(0, CHUNK)]], vbuf, sem2).wait()
            pltpu.make_async_copy(msgbuf, acc.at[lidx], sem3).wait()
            pltpu.make_async_copy(zvals, accz.at[zidx], sem4).wait()
            plsc.subcore_barrier()

            nb = s * NPS
            pltpu.sync_copy(accz.at[pl.ds(s * 4 * NPS, 4 * NPS)], zstage)

            def norm_body(j, carry):
                rb = nb + j * BROW
                pltpu.sync_copy(acc.at[pl.ds(rb, BROW)], stage)

                def grp_body(g, carry2):
                    lnode = j * BROW + g * 16 + lane
                    nl = g * 16 + lane
                    for h in range(4):
                        o = h * DH
                        zv = plsc.load_gather(zstage, [lnode * 4 + h])
                        rcp = 1.0 / (zv + EPS)
                        for d in range(DH):
                            col = jnp.full((16,), o + d, jnp.int32)
                            wv = plsc.load_gather(stage, [nl, col])
                            plsc.store_scatter(outb, [nl, col], wv * rcp)
                    return carry2

                lax.fori_loop(0, BROW // 16, grp_body, 0)
                pltpu.sync_copy(outb,
                                out_h.at[pl.ds(nlo + rb, BROW),
                                         pl.ds(coff, HALF)])
                return carry

            lax.fori_loop(0, NBLK, norm_body, 0)

        run_pass(0)
        for p in range(1, NPASS):
            plsc.subcore_barrier()
            zero_accs()
            plsc.subcore_barrier()
            run_pass(p)

    return attn(qh, kh, vh, src, dst, zrows, zzrows)


def kernel(q, k, v, edge_index):
    q2 = q.reshape(N, 2, HALF).transpose(1, 0, 2)
    k2 = k.reshape(N, 2, HALF).transpose(1, 0, 2)
    v2 = v.reshape(N, 2, HALF).transpose(1, 0, 2)
    pad = jnp.zeros((IDXB * CHUNK,), jnp.int32)
    src = jnp.concatenate([edge_index[0].astype(jnp.int32), pad])
    dst = jnp.concatenate([edge_index[1].astype(jnp.int32), pad])
    zrows = jnp.zeros((NPS, HALF), jnp.float32)
    zzrows = jnp.zeros((4 * NPS,), jnp.float32)
    out2, _ = _sc_attention(q2, k2, v2, src, dst, zrows, zzrows)
    return out2[:N].reshape(1, N, 2 * HALF)
